# Initial kernel scaffold; baseline (speedup 1.0000x reference)
import jax
import jax.numpy as jnp
from jax.experimental import pallas as pl


def kernel(feat, edge_index, adj_values, W):
    raise NotImplementedError("write your pallas kernel here")


# speedup vs baseline: 3.9309x; 3.9309x over previous
import jax
import jax.numpy as jnp
from jax import lax
from jax.experimental import pallas as pl
from jax.experimental.pallas import tpu as pltpu
from jax.experimental.pallas import tpu_sc as plsc

N_NODES = 10000
N_EDGES = 320000
IN_FEAT = 128
OUT_FEAT = 128

NC = 2
NS = 16
NW = NC * NS
CH = 128
LANES = 16
N_PAD = 10240
ROWS_PER_TILE = N_PAD // NS
T_EDGES = -(-N_EDGES // (NW * CH)) * CH
E_PAD = T_EDGES * NW
N_CHUNKS = T_EDGES // CH


def _mm_body(f_ref, w_ref, o_ref):
    o_ref[...] = jnp.dot(f_ref[...], w_ref[...],
                         preferred_element_type=jnp.float32)


def _matmul(feat, W):
    m = feat.shape[0]
    bm = 1000
    return pl.pallas_call(
        _mm_body,
        grid=(m // bm,),
        in_specs=[
            pl.BlockSpec((bm, IN_FEAT), lambda i: (i, 0)),
            pl.BlockSpec((IN_FEAT, OUT_FEAT), lambda i: (0, 0)),
        ],
        out_specs=pl.BlockSpec((bm, OUT_FEAT), lambda i: (i, 0)),
        out_shape=jax.ShapeDtypeStruct((m, OUT_FEAT), jnp.float32),
    )(feat, W)


def _add_body(p_ref, o_ref):
    o_ref[...] = p_ref[0] + p_ref[1]


def _add_partials(partials):
    n = N_NODES
    bm = 1000
    return pl.pallas_call(
        _add_body,
        grid=(n // bm,),
        in_specs=[pl.BlockSpec((NC, bm, OUT_FEAT), lambda i: (0, i, 0))],
        out_specs=pl.BlockSpec((bm, OUT_FEAT), lambda i: (i, 0)),
        out_shape=jax.ShapeDtypeStruct((n, OUT_FEAT), jnp.float32),
    )(partials)


def _spmm_body(emb_hbm, src_hbm, dst_hbm, val_hbm, zeros_hbm, out_hbm,
               src_c, dst_c, val_c, rows, acc, gsem):
    c = lax.axis_index("c")
    s = lax.axis_index("s")
    base_rows = s * ROWS_PER_TILE

    pltpu.sync_copy(zeros_hbm, acc.at[pl.ds(base_rows, ROWS_PER_TILE)])
    plsc.subcore_barrier()

    tile_base = (c * NS + s) * T_EDGES

    @pl.loop(0, N_CHUNKS)
    def _chunk(i):
        base = tile_base + i * CH
        pltpu.sync_copy(src_hbm.at[pl.ds(base, CH)], src_c)
        pltpu.sync_copy(dst_hbm.at[pl.ds(base, CH)], dst_c)
        pltpu.sync_copy(val_hbm.at[pl.ds(base, CH)], val_c)
        pltpu.async_copy(emb_hbm.at[src_c], rows, gsem).wait()

        @pl.loop(0, CH // LANES)
        def _scale(g):
            vvec = val_c[pl.ds(g * LANES, LANES)]
            for j in range(LANES):
                v = vvec[j]
                e = g * LANES + j
                for d in range(OUT_FEAT // LANES):
                    sl = pl.ds(d * LANES, LANES)
                    rows[e, sl] = rows[e, sl] * v

        pltpu.sync_copy(rows, acc.at[dst_c], add=True)

    plsc.subcore_barrier()
    pltpu.sync_copy(acc.at[pl.ds(base_rows, ROWS_PER_TILE)],
                    out_hbm.at[c, pl.ds(base_rows, ROWS_PER_TILE)])


_sc_mesh = plsc.VectorSubcoreMesh(core_axis_name="c", subcore_axis_name="s")

_spmm = pl.kernel(
    _spmm_body,
    out_type=jax.ShapeDtypeStruct((NC, N_PAD, OUT_FEAT), jnp.float32),
    mesh=_sc_mesh,
    scratch_types=[
        pltpu.VMEM((CH,), jnp.int32),
        pltpu.VMEM((CH,), jnp.int32),
        pltpu.VMEM((CH,), jnp.float32),
        pltpu.VMEM((CH, OUT_FEAT), jnp.float32),
        pltpu.VMEM_SHARED((N_PAD, OUT_FEAT), jnp.float32),
        pltpu.SemaphoreType.DMA,
    ],
)


def kernel(feat, edge_index, adj_values, W):
    emb = _matmul(feat, W)

    src = edge_index[1].astype(jnp.int32)
    dst = edge_index[0].astype(jnp.int32)
    vals = adj_values.astype(jnp.float32)
    pad = E_PAD - N_EDGES
    src = jnp.concatenate([src, jnp.zeros((pad,), jnp.int32)])
    dst = jnp.concatenate([dst, jnp.zeros((pad,), jnp.int32)])
    vals = jnp.concatenate([vals, jnp.zeros((pad,), jnp.float32)])
    zeros = jnp.zeros((ROWS_PER_TILE, OUT_FEAT), jnp.float32)

    partials = _spmm(emb, src, dst, vals, zeros)
    x = _add_partials(partials)
    return (emb, x)
